# baseline (speedup 1.0000x reference)
import functools

import jax
import jax.numpy as jnp
from jax import lax
from jax.experimental import pallas as pl
from jax.experimental.pallas import tpu as pltpu
from jax.experimental.pallas import tpu_sc as plsc

N = 10000
E = 320000
D_IN = 128
HID = 32
HEADS = 4
D_OUT = 64

NC = 2
NS = 16
L = 16
C = 80
EPT = E // (NC * NS)
NCH = EPT // C
NPAD = 10240
RPT = NPAD // NS
ZR = 32
AW = 16
SW = 16


def _dense_in_body(x_ref, w_ref, a_ref, h_ref, t_ref):
    h = jnp.dot(x_ref[...], w_ref[...], preferred_element_type=jnp.float32)
    h_ref[...] = h
    t_ref[...] = jnp.dot(h, a_ref[...], preferred_element_type=jnp.float32)


def _dense_in(x, W, A):
    return pl.pallas_call(
        _dense_in_body,
        out_shape=[
            jax.ShapeDtypeStruct((N, W.shape[1]), jnp.float32),
            jax.ShapeDtypeStruct((N, A.shape[1]), jnp.float32),
        ],
    )(x, W, A)


def _dense_mid_body(acc_ref, s_ref, b_ref, g_ref, be_ref, w_ref, a_ref,
                    p_ref, h_ref, t_ref):
    agg = acc_ref[0, pl.ds(0, N)] + acc_ref[1, pl.ds(0, N)]
    s4 = (s_ref[0, pl.ds(0, N), 0:HEADS]
          + s_ref[1, pl.ds(0, N), 0:HEADS])
    rec = 1.0 / (s4 + 1e-16)
    x1 = agg * jnp.dot(rec, p_ref[...], preferred_element_type=jnp.float32)
    x1 = x1 + b_ref[...]
    m = jnp.mean(x1, axis=0, keepdims=True)
    v = jnp.mean((x1 - m) ** 2, axis=0, keepdims=True)
    y = (x1 - m) * lax.rsqrt(v + 1e-5) * g_ref[...] + be_ref[...]
    r = jnp.maximum(y, 0.0)
    h = jnp.dot(r, w_ref[...], preferred_element_type=jnp.float32)
    h_ref[...] = h
    t_ref[...] = jnp.dot(h, a_ref[...], preferred_element_type=jnp.float32)


def _dense_mid(acc, s, b, g, be, W, A, P):
    return pl.pallas_call(
        _dense_mid_body,
        out_shape=[
            jax.ShapeDtypeStruct((N, W.shape[1]), jnp.float32),
            jax.ShapeDtypeStruct((N, A.shape[1]), jnp.float32),
        ],
    )(acc, s, b, g, be, W, A, P)


def _dense_out_body(acc_ref, s_ref, b_ref, o_ref):
    agg = acc_ref[0, pl.ds(0, N)] + acc_ref[1, pl.ds(0, N)]
    s1 = s_ref[0, pl.ds(0, N), 0:1] + s_ref[1, pl.ds(0, N), 0:1]
    x1 = agg / (s1 + 1e-16) + b_ref[...]
    mx = jnp.max(x1, axis=1, keepdims=True)
    e = jnp.exp(x1 - mx)
    lse = jnp.log(jnp.sum(e, axis=1, keepdims=True)) + mx
    o_ref[...] = x1 - lse


def _dense_out(acc, s, b):
    return pl.pallas_call(
        _dense_out_body,
        out_shape=jax.ShapeDtypeStruct((N, D_OUT), jnp.float32),
    )(acc, s, b)


def _edge_body(D, H, src_hbm, dst_hbm, h_hbm, ts_hbm, td_hbm, acc_out, s_out,
               srcA, dstA, rowsA, asA, adA, srcB, dstB, rowsB, asB, adB,
               w_v, zrow_v, zs_v, acc_sh, s_sh,
               sAr, sAa, sAb, sBr, sBa, sBb):
    cid = lax.axis_index("c")
    sid = lax.axis_index("s")
    wid = cid * NS + sid
    SEG = D // H
    QH = SEG // L
    zvec = jnp.zeros((L,), jnp.float32)

    bufA = (srcA, dstA, rowsA, asA, adA, sAr, sAa, sAb)
    bufB = (srcB, dstB, rowsB, asB, adB, sBr, sBa, sBb)

    def _zrow(i, c):
        for q in range(D // L):
            zrow_v[i, pl.ds(q * L, L)] = zvec
        zs_v[i, :] = zvec
        return c
    lax.fori_loop(0, ZR, _zrow, 0)

    for k in range(RPT // ZR):
        pltpu.sync_copy(zrow_v, acc_sh.at[pl.ds(sid * RPT + k * ZR, ZR)])
        pltpu.sync_copy(zs_v, s_sh.at[pl.ds(sid * RPT + k * ZR, ZR)])
    plsc.subcore_barrier()

    def issue(c, buf):
        src_v, dst_v, rows_v, as_v, ad_v, sem_r, sem_a, sem_b = buf
        base = wid * EPT + c * C
        pltpu.sync_copy(src_hbm.at[pl.ds(base, C)], src_v)
        pltpu.sync_copy(dst_hbm.at[pl.ds(base, C)], dst_v)
        pltpu.async_copy(h_hbm.at[src_v], rows_v, sem_r)
        pltpu.async_copy(ts_hbm.at[src_v], as_v, sem_a)
        pltpu.async_copy(td_hbm.at[dst_v], ad_v, sem_b)

    def process(buf):
        src_v, dst_v, rows_v, as_v, ad_v, sem_r, sem_a, sem_b = buf
        pltpu.make_async_copy(ts_hbm.at[src_v], as_v, sem_a).wait()
        pltpu.make_async_copy(td_hbm.at[dst_v], ad_v, sem_b).wait()
        pltpu.make_async_copy(h_hbm.at[src_v], rows_v, sem_r).wait()

        def _es(e, cc):
            lg = as_v[e, :] + ad_v[e, :]
            lg = jnp.maximum(lg, 0.2 * lg)
            w_v[e, :] = jnp.exp(lg)
            ev = jnp.full((L,), e, jnp.int32)
            for h in range(H):
                hv = jnp.full((L,), h, jnp.int32)
                wv = plsc.load_gather(w_v, [ev, hv])
                for q in range(QH):
                    col = h * SEG + q * L
                    rows_v[e, pl.ds(col, L)] = rows_v[e, pl.ds(col, L)] * wv
            return cc
        lax.fori_loop(0, C, _es, 0)

        pltpu.sync_copy(rows_v, acc_sh.at[dst_v], add=True)
        pltpu.sync_copy(w_v, s_sh.at[dst_v], add=True)

    issue(0, bufA)

    def _pair(i, c):
        issue(2 * i + 1, bufB)
        process(bufA)
        issue(2 * i + 2, bufA)
        process(bufB)
        return c
    lax.fori_loop(0, (NCH - 1) // 2, _pair, 0)
    process(bufA)

    plsc.subcore_barrier()

    pltpu.sync_copy(acc_sh.at[pl.ds(sid * RPT, RPT)],
                    acc_out.at[cid, pl.ds(sid * RPT, RPT)])
    pltpu.sync_copy(s_sh.at[pl.ds(sid * RPT, RPT)],
                    s_out.at[cid, pl.ds(sid * RPT, RPT)])


@functools.lru_cache(maxsize=None)
def _make_edge(D, H):
    mesh = plsc.VectorSubcoreMesh(core_axis_name="c", subcore_axis_name="s",
                                  num_cores=NC, num_subcores=NS)
    return pl.kernel(
        functools.partial(_edge_body, D, H),
        compiler_params=pltpu.CompilerParams(needs_layout_passes=False,
                                             use_tc_tiling_on_sc=False),
        out_type=[
            jax.ShapeDtypeStruct((NC, NPAD, D), jnp.float32),
            jax.ShapeDtypeStruct((NC, NPAD, SW), jnp.float32),
        ],
        mesh=mesh,
        scratch_types=[
            pltpu.VMEM((C,), jnp.int32),
            pltpu.VMEM((C,), jnp.int32),
            pltpu.VMEM((C, D), jnp.float32),
            pltpu.VMEM((C, AW), jnp.float32),
            pltpu.VMEM((C, AW), jnp.float32),
            pltpu.VMEM((C,), jnp.int32),
            pltpu.VMEM((C,), jnp.int32),
            pltpu.VMEM((C, D), jnp.float32),
            pltpu.VMEM((C, AW), jnp.float32),
            pltpu.VMEM((C, AW), jnp.float32),
            pltpu.VMEM((C, SW), jnp.float32),
            pltpu.VMEM((ZR, D), jnp.float32),
            pltpu.VMEM((ZR, SW), jnp.float32),
            pltpu.VMEM_SHARED((NPAD, D), jnp.float32),
            pltpu.VMEM_SHARED((NPAD, SW), jnp.float32),
            pltpu.SemaphoreType.DMA,
            pltpu.SemaphoreType.DMA,
            pltpu.SemaphoreType.DMA,
            pltpu.SemaphoreType.DMA,
            pltpu.SemaphoreType.DMA,
            pltpu.SemaphoreType.DMA,
        ],
    )


def kernel(x, edge_index, W1, as1, ad1, b1, g1, be1, W2, as2, ad2, b2, g2,
           be2, W3, as3, ad3, b3):
    f32 = jnp.float32
    src = edge_index[0].astype(jnp.int32)
    dst = edge_index[1].astype(jnp.int32)

    eye = jnp.eye(HEADS, dtype=f32)
    zpad = jnp.zeros((HEADS * HID, AW - HEADS), f32)
    A1 = jnp.concatenate(
        [(eye[:, None, :] * as1[:, :, None]).reshape(HEADS * HID, HEADS), zpad,
         (eye[:, None, :] * ad1[:, :, None]).reshape(HEADS * HID, HEADS), zpad],
        axis=1)
    A2 = jnp.concatenate(
        [(eye[:, None, :] * as2[:, :, None]).reshape(HEADS * HID, HEADS), zpad,
         (eye[:, None, :] * ad2[:, :, None]).reshape(HEADS * HID, HEADS), zpad],
        axis=1)
    zpad3 = jnp.zeros((D_OUT, AW - 1), f32)
    A3 = jnp.concatenate([as3[0][:, None], zpad3, ad3[0][:, None], zpad3],
                         axis=1)
    P = jnp.repeat(eye, HID, axis=1)

    b1r = b1.reshape(1, -1)
    g1r = g1.reshape(1, -1)
    be1r = be1.reshape(1, -1)
    b2r = b2.reshape(1, -1)
    g2r = g2.reshape(1, -1)
    be2r = be2.reshape(1, -1)
    b3r = b3.reshape(1, -1)

    edge128 = _make_edge(HEADS * HID, HEADS)
    edge64 = _make_edge(D_OUT, 1)

    h1, t1 = _dense_in(x, W1, A1)
    acc, s = edge128(src, dst, h1, t1[:, :AW], t1[:, AW:])
    h2, t2 = _dense_mid(acc, s, b1r, g1r, be1r, W2, A2, P)
    acc, s = edge128(src, dst, h2, t2[:, :AW], t2[:, AW:])
    h3, t3 = _dense_mid(acc, s, b2r, g2r, be2r, W3, A3, P)
    acc, s = edge64(src, dst, h3, t3[:, :AW], t3[:, AW:])
    return _dense_out(acc, s, b3r)


# speedup vs baseline: 1.2060x; 1.2060x over previous
import functools

import jax
import jax.numpy as jnp
from jax import lax
from jax.experimental import pallas as pl
from jax.experimental.pallas import tpu as pltpu
from jax.experimental.pallas import tpu_sc as plsc

N = 10000
E = 320000
D_IN = 128
HID = 32
HEADS = 4
D_OUT = 64

NC = 2
NS = 16
L = 16
C = 80
EPT = E // (NC * NS)
SUP = 2000
NSUP = EPT // SUP
NCHS = SUP // C
NPAD = 10112
RPT = NPAD // NS
ZR = 8
AW = 16
SW = 16


def _dense_in_body(x_ref, w_ref, a_ref, h_ref, t_ref):
    h = jnp.dot(x_ref[...], w_ref[...], preferred_element_type=jnp.float32)
    h_ref[...] = h
    t_ref[...] = jnp.dot(h, a_ref[...], preferred_element_type=jnp.float32)


def _dense_in(x, W, A):
    return pl.pallas_call(
        _dense_in_body,
        out_shape=[
            jax.ShapeDtypeStruct((N, W.shape[1]), jnp.float32),
            jax.ShapeDtypeStruct((N, A.shape[1]), jnp.float32),
        ],
    )(x, W, A)


def _dense_mid_body(acc_ref, s_ref, b_ref, g_ref, be_ref, w_ref, a_ref,
                    p_ref, h_ref, t_ref):
    agg = acc_ref[0, pl.ds(0, N)] + acc_ref[1, pl.ds(0, N)]
    s4 = (s_ref[0, pl.ds(0, N), 0:HEADS]
          + s_ref[1, pl.ds(0, N), 0:HEADS])
    rec = 1.0 / (s4 + 1e-16)
    x1 = agg * jnp.dot(rec, p_ref[...], preferred_element_type=jnp.float32)
    x1 = x1 + b_ref[...]
    m = jnp.mean(x1, axis=0, keepdims=True)
    v = jnp.mean((x1 - m) ** 2, axis=0, keepdims=True)
    y = (x1 - m) * lax.rsqrt(v + 1e-5) * g_ref[...] + be_ref[...]
    r = jnp.maximum(y, 0.0)
    h = jnp.dot(r, w_ref[...], preferred_element_type=jnp.float32)
    h_ref[...] = h
    t_ref[...] = jnp.dot(h, a_ref[...], preferred_element_type=jnp.float32)


def _dense_mid(acc, s, b, g, be, W, A, P):
    return pl.pallas_call(
        _dense_mid_body,
        out_shape=[
            jax.ShapeDtypeStruct((N, W.shape[1]), jnp.float32),
            jax.ShapeDtypeStruct((N, A.shape[1]), jnp.float32),
        ],
    )(acc, s, b, g, be, W, A, P)


def _dense_out_body(acc_ref, s_ref, b_ref, o_ref):
    agg = acc_ref[0, pl.ds(0, N)] + acc_ref[1, pl.ds(0, N)]
    s1 = s_ref[0, pl.ds(0, N), 0:1] + s_ref[1, pl.ds(0, N), 0:1]
    x1 = agg / (s1 + 1e-16) + b_ref[...]
    mx = jnp.max(x1, axis=1, keepdims=True)
    e = jnp.exp(x1 - mx)
    lse = jnp.log(jnp.sum(e, axis=1, keepdims=True)) + mx
    o_ref[...] = x1 - lse


def _dense_out(acc, s, b):
    return pl.pallas_call(
        _dense_out_body,
        out_shape=jax.ShapeDtypeStruct((N, D_OUT), jnp.float32),
    )(acc, s, b)


def _edge_body(D, H, src_hbm, dst_hbm, h_hbm, ts_hbm, td_hbm, acc_out, s_out,
               srcS0, dstS0, srcS1, dstS1,
               rowsA, asA, adA, rowsB, asB, adB,
               w_v, zrow_v, zs_v, acc_sh, s_sh,
               sAr, sAa, sAb, sBr, sBa, sBb, s_i):
    cid = lax.axis_index("c")
    sid = lax.axis_index("s")
    wid = cid * NS + sid
    SEG = D // H
    QH = SEG // L
    zvec = jnp.zeros((L,), jnp.float32)

    idxbuf = ((srcS0, dstS0), (srcS1, dstS1))

    def _zrow(i, c):
        for q in range(D // L):
            zrow_v[i, pl.ds(q * L, L)] = zvec
        zs_v[i, :] = zvec
        return c
    lax.fori_loop(0, ZR, _zrow, 0)

    def _zcp(k, c):
        pltpu.sync_copy(zrow_v, acc_sh.at[pl.ds(sid * RPT + k * ZR, ZR)])
        pltpu.sync_copy(zs_v, s_sh.at[pl.ds(sid * RPT + k * ZR, ZR)])
        return c
    lax.fori_loop(0, RPT // ZR, _zcp, 0)
    plsc.subcore_barrier()

    def idx_issue(s, srcS, dstS):
        base = wid * EPT + s * SUP
        pltpu.async_copy(src_hbm.at[pl.ds(base, SUP)], srcS, s_i)
        pltpu.async_copy(dst_hbm.at[pl.ds(base, SUP)], dstS, s_i)

    def idx_wait(srcS, dstS):
        pltpu.make_async_copy(src_hbm.at[pl.ds(0, SUP)], srcS, s_i).wait()
        pltpu.make_async_copy(dst_hbm.at[pl.ds(0, SUP)], dstS, s_i).wait()

    def issue(j, srcS, dstS, buf):
        rows_v, as_v, ad_v, sem_r, sem_a, sem_b = buf
        sv = srcS.at[pl.ds(j * C, C)]
        dv = dstS.at[pl.ds(j * C, C)]
        pltpu.async_copy(h_hbm.at[sv], rows_v, sem_r)
        pltpu.async_copy(ts_hbm.at[sv], as_v, sem_a)
        pltpu.async_copy(td_hbm.at[dv], ad_v, sem_b)

    def process(j, srcS, dstS, buf):
        rows_v, as_v, ad_v, sem_r, sem_a, sem_b = buf
        sv = srcS.at[pl.ds(j * C, C)]
        dv = dstS.at[pl.ds(j * C, C)]
        pltpu.make_async_copy(ts_hbm.at[sv], as_v, sem_a).wait()
        pltpu.make_async_copy(td_hbm.at[dv], ad_v, sem_b).wait()
        pltpu.make_async_copy(h_hbm.at[sv], rows_v, sem_r).wait()

        def _es(e, cc):
            lg = as_v[e, :] + ad_v[e, :]
            lg = jnp.maximum(lg, 0.2 * lg)
            w_v[e, :] = jnp.exp(lg)
            ev = jnp.full((L,), e, jnp.int32)
            for h in range(H):
                hv = jnp.full((L,), h, jnp.int32)
                wv = plsc.load_gather(w_v, [ev, hv])
                for q in range(QH):
                    col = h * SEG + q * L
                    rows_v[e, pl.ds(col, L)] = rows_v[e, pl.ds(col, L)] * wv
            return cc
        lax.fori_loop(0, C, _es, 0)

        pltpu.sync_copy(rows_v, acc_sh.at[dv], add=True)
        pltpu.sync_copy(w_v, s_sh.at[dv], add=True)

    bufA = (rowsA, asA, adA, sAr, sAa, sAb)
    bufB = (rowsB, asB, adB, sBr, sBa, sBb)
    idx_issue(0, *idxbuf[0])
    idx_wait(*idxbuf[0])
    for s in range(NSUP):
        srcS, dstS = idxbuf[s % 2]
        if s + 1 < NSUP:
            idx_issue(s + 1, *idxbuf[(s + 1) % 2])
        issue(0, srcS, dstS, bufA)

        def _pair(i, c):
            issue(2 * i + 1, srcS, dstS, bufB)
            process(2 * i, srcS, dstS, bufA)
            issue(2 * i + 2, srcS, dstS, bufA)
            process(2 * i + 1, srcS, dstS, bufB)
            return c
        lax.fori_loop(0, (NCHS - 1) // 2, _pair, 0)
        process(NCHS - 1, srcS, dstS, bufA)
        if s + 1 < NSUP:
            idx_wait(*idxbuf[(s + 1) % 2])

    plsc.subcore_barrier()

    pltpu.sync_copy(acc_sh.at[pl.ds(sid * RPT, RPT)],
                    acc_out.at[cid, pl.ds(sid * RPT, RPT)])
    pltpu.sync_copy(s_sh.at[pl.ds(sid * RPT, RPT)],
                    s_out.at[cid, pl.ds(sid * RPT, RPT)])


@functools.lru_cache(maxsize=None)
def _make_edge(D, H):
    mesh = plsc.VectorSubcoreMesh(core_axis_name="c", subcore_axis_name="s",
                                  num_cores=NC, num_subcores=NS)
    return pl.kernel(
        functools.partial(_edge_body, D, H),
        compiler_params=pltpu.CompilerParams(needs_layout_passes=False,
                                             use_tc_tiling_on_sc=False),
        out_type=[
            jax.ShapeDtypeStruct((NC, NPAD, D), jnp.float32),
            jax.ShapeDtypeStruct((NC, NPAD, SW), jnp.float32),
        ],
        mesh=mesh,
        scratch_types=[
            pltpu.VMEM((SUP,), jnp.int32),
            pltpu.VMEM((SUP,), jnp.int32),
            pltpu.VMEM((SUP,), jnp.int32),
            pltpu.VMEM((SUP,), jnp.int32),
            pltpu.VMEM((C, D), jnp.float32),
            pltpu.VMEM((C, AW), jnp.float32),
            pltpu.VMEM((C, AW), jnp.float32),
            pltpu.VMEM((C, D), jnp.float32),
            pltpu.VMEM((C, AW), jnp.float32),
            pltpu.VMEM((C, AW), jnp.float32),
            pltpu.VMEM((C, SW), jnp.float32),
            pltpu.VMEM((ZR, D), jnp.float32),
            pltpu.VMEM((ZR, SW), jnp.float32),
            pltpu.VMEM_SHARED((NPAD, D), jnp.float32),
            pltpu.VMEM_SHARED((NPAD, SW), jnp.float32),
            pltpu.SemaphoreType.DMA,
            pltpu.SemaphoreType.DMA,
            pltpu.SemaphoreType.DMA,
            pltpu.SemaphoreType.DMA,
            pltpu.SemaphoreType.DMA,
            pltpu.SemaphoreType.DMA,
            pltpu.SemaphoreType.DMA,
        ],
    )


def kernel(x, edge_index, W1, as1, ad1, b1, g1, be1, W2, as2, ad2, b2, g2,
           be2, W3, as3, ad3, b3):
    f32 = jnp.float32
    src = edge_index[0].astype(jnp.int32)
    dst = edge_index[1].astype(jnp.int32)

    eye = jnp.eye(HEADS, dtype=f32)
    zpad = jnp.zeros((HEADS * HID, AW - HEADS), f32)
    A1 = jnp.concatenate(
        [(eye[:, None, :] * as1[:, :, None]).reshape(HEADS * HID, HEADS), zpad,
         (eye[:, None, :] * ad1[:, :, None]).reshape(HEADS * HID, HEADS), zpad],
        axis=1)
    A2 = jnp.concatenate(
        [(eye[:, None, :] * as2[:, :, None]).reshape(HEADS * HID, HEADS), zpad,
         (eye[:, None, :] * ad2[:, :, None]).reshape(HEADS * HID, HEADS), zpad],
        axis=1)
    zpad3 = jnp.zeros((D_OUT, AW - 1), f32)
    A3 = jnp.concatenate([as3[0][:, None], zpad3, ad3[0][:, None], zpad3],
                         axis=1)
    P = jnp.repeat(eye, HID, axis=1)

    b1r = b1.reshape(1, -1)
    g1r = g1.reshape(1, -1)
    be1r = be1.reshape(1, -1)
    b2r = b2.reshape(1, -1)
    g2r = g2.reshape(1, -1)
    be2r = be2.reshape(1, -1)
    b3r = b3.reshape(1, -1)

    edge128 = _make_edge(HEADS * HID, HEADS)
    edge64 = _make_edge(D_OUT, 1)

    h1, t1 = _dense_in(x, W1, A1)
    acc, s = edge128(src, dst, h1, t1[:, :AW], t1[:, AW:])
    h2, t2 = _dense_mid(acc, s, b1r, g1r, be1r, W2, A2, P)
    acc, s = edge128(src, dst, h2, t2[:, :AW], t2[:, AW:])
    h3, t3 = _dense_mid(acc, s, b2r, g2r, be2r, W3, A3, P)
    acc, s = edge64(src, dst, h3, t3[:, :AW], t3[:, AW:])
    return _dense_out(acc, s, b3r)


# speedup vs baseline: 1.2171x; 1.0092x over previous
import functools

import jax
import jax.numpy as jnp
from jax import lax
from jax.experimental import pallas as pl
from jax.experimental.pallas import tpu as pltpu
from jax.experimental.pallas import tpu_sc as plsc

N = 10000
E = 320000
D_IN = 128
HID = 32
HEADS = 4
D_OUT = 64

NC = 2
NS = 16
L = 16
C = 80
EPT = E // (NC * NS)
SUP = 2000
NSUP = EPT // SUP
NCHS = SUP // C
NPAD = 10112
RPT = NPAD // NS
ZR = 8
AW = 16
SW = 16


def _dense_in_body(x_ref, w_ref, a_ref, h_ref, t_ref):
    h = jnp.dot(x_ref[...], w_ref[...], preferred_element_type=jnp.float32)
    h_ref[...] = h
    t_ref[...] = jnp.dot(h, a_ref[...], preferred_element_type=jnp.float32)


def _dense_in(x, W, A):
    return pl.pallas_call(
        _dense_in_body,
        out_shape=[
            jax.ShapeDtypeStruct((N, W.shape[1]), jnp.float32),
            jax.ShapeDtypeStruct((N, A.shape[1]), jnp.float32),
        ],
    )(x, W, A)


def _dense_mid_body(acc_ref, s_ref, b_ref, g_ref, be_ref, w_ref, a_ref,
                    p_ref, h_ref, t_ref):
    agg = acc_ref[0, pl.ds(0, N)] + acc_ref[1, pl.ds(0, N)]
    s4 = (s_ref[0, pl.ds(0, N), 0:HEADS]
          + s_ref[1, pl.ds(0, N), 0:HEADS])
    rec = 1.0 / (s4 + 1e-16)
    x1 = agg * jnp.dot(rec, p_ref[...], preferred_element_type=jnp.float32)
    x1 = x1 + b_ref[...]
    m = jnp.mean(x1, axis=0, keepdims=True)
    v = jnp.mean((x1 - m) ** 2, axis=0, keepdims=True)
    y = (x1 - m) * lax.rsqrt(v + 1e-5) * g_ref[...] + be_ref[...]
    r = jnp.maximum(y, 0.0)
    h = jnp.dot(r, w_ref[...], preferred_element_type=jnp.float32)
    h_ref[...] = h
    t_ref[...] = jnp.dot(h, a_ref[...], preferred_element_type=jnp.float32)


def _dense_mid(acc, s, b, g, be, W, A, P):
    return pl.pallas_call(
        _dense_mid_body,
        out_shape=[
            jax.ShapeDtypeStruct((N, W.shape[1]), jnp.float32),
            jax.ShapeDtypeStruct((N, A.shape[1]), jnp.float32),
        ],
    )(acc, s, b, g, be, W, A, P)


def _dense_out_body(acc_ref, s_ref, b_ref, o_ref):
    agg = acc_ref[0, pl.ds(0, N)] + acc_ref[1, pl.ds(0, N)]
    s1 = s_ref[0, pl.ds(0, N), 0:1] + s_ref[1, pl.ds(0, N), 0:1]
    x1 = agg / (s1 + 1e-16) + b_ref[...]
    mx = jnp.max(x1, axis=1, keepdims=True)
    e = jnp.exp(x1 - mx)
    lse = jnp.log(jnp.sum(e, axis=1, keepdims=True)) + mx
    o_ref[...] = x1 - lse


def _dense_out(acc, s, b):
    return pl.pallas_call(
        _dense_out_body,
        out_shape=jax.ShapeDtypeStruct((N, D_OUT), jnp.float32),
    )(acc, s, b)


def _edge_body(D, H, src_hbm, dst_hbm, h_hbm, ts_hbm, td_hbm, acc_out, s_out,
               srcS0, dstS0, srcS1, dstS1,
               rowsA, asA, adA, rowsB, asB, adB,
               w_v, zrow_v, zs_v, acc_sh, s_sh,
               sAr, sAa, sAb, sBr, sBa, sBb, s_i):
    cid = lax.axis_index("c")
    sid = lax.axis_index("s")
    wid = cid * NS + sid
    SEG = D // H
    QH = SEG // L
    zvec = jnp.zeros((L,), jnp.float32)

    idxbuf = ((srcS0, dstS0), (srcS1, dstS1))

    def _zrow(i, c):
        for q in range(D // L):
            zrow_v[i, pl.ds(q * L, L)] = zvec
        zs_v[i, :] = zvec
        return c
    lax.fori_loop(0, ZR, _zrow, 0)

    def _zcp(k, c):
        pltpu.sync_copy(zrow_v, acc_sh.at[pl.ds(sid * RPT + k * ZR, ZR)])
        pltpu.sync_copy(zs_v, s_sh.at[pl.ds(sid * RPT + k * ZR, ZR)])
        return c
    lax.fori_loop(0, RPT // ZR, _zcp, 0)
    plsc.subcore_barrier()

    def idx_issue(s, srcS, dstS):
        base = wid * EPT + s * SUP
        pltpu.async_copy(src_hbm.at[pl.ds(base, SUP)], srcS, s_i)
        pltpu.async_copy(dst_hbm.at[pl.ds(base, SUP)], dstS, s_i)

    def idx_wait(srcS, dstS):
        pltpu.make_async_copy(src_hbm.at[pl.ds(0, SUP)], srcS, s_i).wait()
        pltpu.make_async_copy(dst_hbm.at[pl.ds(0, SUP)], dstS, s_i).wait()

    def issue(j, srcS, dstS, buf):
        rows_v, as_v, ad_v, sem_r, sem_a, sem_b = buf
        sv = srcS.at[pl.ds(j * C, C)]
        dv = dstS.at[pl.ds(j * C, C)]
        pltpu.async_copy(h_hbm.at[sv], rows_v, sem_r)
        pltpu.async_copy(ts_hbm.at[sv], as_v, sem_a)
        pltpu.async_copy(td_hbm.at[dv], ad_v, sem_b)

    def process(j, srcS, dstS, buf):
        rows_v, as_v, ad_v, sem_r, sem_a, sem_b = buf
        sv = srcS.at[pl.ds(j * C, C)]
        dv = dstS.at[pl.ds(j * C, C)]
        pltpu.make_async_copy(ts_hbm.at[sv], as_v, sem_a).wait()
        pltpu.make_async_copy(td_hbm.at[dv], ad_v, sem_b).wait()
        pltpu.make_async_copy(h_hbm.at[sv], rows_v, sem_r).wait()

        def _es(i, cc):
            for u in range(2):
                e = 2 * i + u
                lg = as_v[e, :] + ad_v[e, :]
                lg = jnp.maximum(lg, 0.2 * lg)
                w_v[e, :] = jnp.exp(lg)
                ev = jnp.full((L,), e, jnp.int32)
                for h in range(H):
                    hv = jnp.full((L,), h, jnp.int32)
                    wv = plsc.load_gather(w_v, [ev, hv])
                    for q in range(QH):
                        col = h * SEG + q * L
                        rows_v[e, pl.ds(col, L)] = rows_v[e, pl.ds(col, L)] * wv
            return cc
        lax.fori_loop(0, C // 2, _es, 0)

        pltpu.sync_copy(rows_v, acc_sh.at[dv], add=True)
        pltpu.sync_copy(w_v, s_sh.at[dv], add=True)

    bufA = (rowsA, asA, adA, sAr, sAa, sAb)
    bufB = (rowsB, asB, adB, sBr, sBa, sBb)
    idx_issue(0, *idxbuf[0])
    idx_wait(*idxbuf[0])
    for s in range(NSUP):
        srcS, dstS = idxbuf[s % 2]
        if s + 1 < NSUP:
            idx_issue(s + 1, *idxbuf[(s + 1) % 2])
        issue(0, srcS, dstS, bufA)

        def _pair(i, c):
            issue(2 * i + 1, srcS, dstS, bufB)
            process(2 * i, srcS, dstS, bufA)
            issue(2 * i + 2, srcS, dstS, bufA)
            process(2 * i + 1, srcS, dstS, bufB)
            return c
        lax.fori_loop(0, (NCHS - 1) // 2, _pair, 0)
        process(NCHS - 1, srcS, dstS, bufA)
        if s + 1 < NSUP:
            idx_wait(*idxbuf[(s + 1) % 2])

    plsc.subcore_barrier()

    pltpu.sync_copy(acc_sh.at[pl.ds(sid * RPT, RPT)],
                    acc_out.at[cid, pl.ds(sid * RPT, RPT)])
    pltpu.sync_copy(s_sh.at[pl.ds(sid * RPT, RPT)],
                    s_out.at[cid, pl.ds(sid * RPT, RPT)])


@functools.lru_cache(maxsize=None)
def _make_edge(D, H):
    mesh = plsc.VectorSubcoreMesh(core_axis_name="c", subcore_axis_name="s",
                                  num_cores=NC, num_subcores=NS)
    return pl.kernel(
        functools.partial(_edge_body, D, H),
        compiler_params=pltpu.CompilerParams(needs_layout_passes=False,
                                             use_tc_tiling_on_sc=False),
        out_type=[
            jax.ShapeDtypeStruct((NC, NPAD, D), jnp.float32),
            jax.ShapeDtypeStruct((NC, NPAD, SW), jnp.float32),
        ],
        mesh=mesh,
        scratch_types=[
            pltpu.VMEM((SUP,), jnp.int32),
            pltpu.VMEM((SUP,), jnp.int32),
            pltpu.VMEM((SUP,), jnp.int32),
            pltpu.VMEM((SUP,), jnp.int32),
            pltpu.VMEM((C, D), jnp.float32),
            pltpu.VMEM((C, AW), jnp.float32),
            pltpu.VMEM((C, AW), jnp.float32),
            pltpu.VMEM((C, D), jnp.float32),
            pltpu.VMEM((C, AW), jnp.float32),
            pltpu.VMEM((C, AW), jnp.float32),
            pltpu.VMEM((C, SW), jnp.float32),
            pltpu.VMEM((ZR, D), jnp.float32),
            pltpu.VMEM((ZR, SW), jnp.float32),
            pltpu.VMEM_SHARED((NPAD, D), jnp.float32),
            pltpu.VMEM_SHARED((NPAD, SW), jnp.float32),
            pltpu.SemaphoreType.DMA,
            pltpu.SemaphoreType.DMA,
            pltpu.SemaphoreType.DMA,
            pltpu.SemaphoreType.DMA,
            pltpu.SemaphoreType.DMA,
            pltpu.SemaphoreType.DMA,
            pltpu.SemaphoreType.DMA,
        ],
    )


def kernel(x, edge_index, W1, as1, ad1, b1, g1, be1, W2, as2, ad2, b2, g2,
           be2, W3, as3, ad3, b3):
    f32 = jnp.float32
    src = edge_index[0].astype(jnp.int32)
    dst = edge_index[1].astype(jnp.int32)

    eye = jnp.eye(HEADS, dtype=f32)
    zpad = jnp.zeros((HEADS * HID, AW - HEADS), f32)
    A1 = jnp.concatenate(
        [(eye[:, None, :] * as1[:, :, None]).reshape(HEADS * HID, HEADS), zpad,
         (eye[:, None, :] * ad1[:, :, None]).reshape(HEADS * HID, HEADS), zpad],
        axis=1)
    A2 = jnp.concatenate(
        [(eye[:, None, :] * as2[:, :, None]).reshape(HEADS * HID, HEADS), zpad,
         (eye[:, None, :] * ad2[:, :, None]).reshape(HEADS * HID, HEADS), zpad],
        axis=1)
    zpad3 = jnp.zeros((D_OUT, AW - 1), f32)
    A3 = jnp.concatenate([as3[0][:, None], zpad3, ad3[0][:, None], zpad3],
                         axis=1)
    P = jnp.repeat(eye, HID, axis=1)

    b1r = b1.reshape(1, -1)
    g1r = g1.reshape(1, -1)
    be1r = be1.reshape(1, -1)
    b2r = b2.reshape(1, -1)
    g2r = g2.reshape(1, -1)
    be2r = be2.reshape(1, -1)
    b3r = b3.reshape(1, -1)

    edge128 = _make_edge(HEADS * HID, HEADS)
    edge64 = _make_edge(D_OUT, 1)

    h1, t1 = _dense_in(x, W1, A1)
    acc, s = edge128(src, dst, h1, t1[:, :AW], t1[:, AW:])
    h2, t2 = _dense_mid(acc, s, b1r, g1r, be1r, W2, A2, P)
    acc, s = edge128(src, dst, h2, t2[:, :AW], t2[:, AW:])
    h3, t3 = _dense_mid(acc, s, b2r, g2r, be2r, W3, A3, P)
    acc, s = edge64(src, dst, h3, t3[:, :AW], t3[:, AW:])
    return _dense_out(acc, s, b3r)


# speedup vs baseline: 1.2174x; 1.0002x over previous
import functools

import jax
import jax.numpy as jnp
from jax import lax
from jax.experimental import pallas as pl
from jax.experimental.pallas import tpu as pltpu
from jax.experimental.pallas import tpu_sc as plsc

N = 10000
E = 320000
D_IN = 128
HID = 32
HEADS = 4
D_OUT = 64

NC = 2
NS = 16
L = 16
C = 80
EPT = E // (NC * NS)
SUP = 2000
NSUP = EPT // SUP
NCHS = SUP // C
NPAD = 10112
RPT = NPAD // NS
ZR = 8
AW = 16
SW = 16


def _dense_in_body(x_ref, w_ref, a_ref, h_ref, t_ref):
    h = jnp.dot(x_ref[...], w_ref[...], preferred_element_type=jnp.float32)
    h_ref[...] = h
    t_ref[...] = jnp.dot(h, a_ref[...], preferred_element_type=jnp.float32)


def _dense_in(x, W, A):
    return pl.pallas_call(
        _dense_in_body,
        out_shape=[
            jax.ShapeDtypeStruct((N, W.shape[1]), jnp.float32),
            jax.ShapeDtypeStruct((N, A.shape[1]), jnp.float32),
        ],
    )(x, W, A)


def _dense_mid_body(acc_ref, s_ref, b_ref, g_ref, be_ref, w_ref, a_ref,
                    p_ref, h_ref, t_ref):
    agg = acc_ref[0, pl.ds(0, N)] + acc_ref[1, pl.ds(0, N)]
    s4 = (s_ref[0, pl.ds(0, N), 0:HEADS]
          + s_ref[1, pl.ds(0, N), 0:HEADS])
    rec = 1.0 / (s4 + 1e-16)
    x1 = agg * jnp.dot(rec, p_ref[...], preferred_element_type=jnp.float32)
    x1 = x1 + b_ref[...]
    m = jnp.mean(x1, axis=0, keepdims=True)
    v = jnp.mean((x1 - m) ** 2, axis=0, keepdims=True)
    y = (x1 - m) * lax.rsqrt(v + 1e-5) * g_ref[...] + be_ref[...]
    r = jnp.maximum(y, 0.0)
    h = jnp.dot(r, w_ref[...], preferred_element_type=jnp.float32)
    h_ref[...] = h
    t_ref[...] = jnp.dot(h, a_ref[...], preferred_element_type=jnp.float32)


def _dense_mid(acc, s, b, g, be, W, A, P):
    return pl.pallas_call(
        _dense_mid_body,
        out_shape=[
            jax.ShapeDtypeStruct((N, W.shape[1]), jnp.float32),
            jax.ShapeDtypeStruct((N, A.shape[1]), jnp.float32),
        ],
    )(acc, s, b, g, be, W, A, P)


def _dense_out_body(acc_ref, s_ref, b_ref, o_ref):
    agg = acc_ref[0, pl.ds(0, N)] + acc_ref[1, pl.ds(0, N)]
    s1 = s_ref[0, pl.ds(0, N), 0:1] + s_ref[1, pl.ds(0, N), 0:1]
    x1 = agg / (s1 + 1e-16) + b_ref[...]
    mx = jnp.max(x1, axis=1, keepdims=True)
    e = jnp.exp(x1 - mx)
    lse = jnp.log(jnp.sum(e, axis=1, keepdims=True)) + mx
    o_ref[...] = x1 - lse


def _dense_out(acc, s, b):
    return pl.pallas_call(
        _dense_out_body,
        out_shape=jax.ShapeDtypeStruct((N, D_OUT), jnp.float32),
    )(acc, s, b)


def _edge_body(D, H, src_hbm, dst_hbm, h_hbm, ts_hbm, td_hbm, acc_out, s_out,
               srcS0, dstS0, srcS1, dstS1,
               rowsA, asA, adA, rowsB, asB, adB,
               w_v, zrow_v, zs_v, acc_sh, s_sh,
               sAr, sAa, sAb, sBr, sBa, sBb, s_i):
    cid = lax.axis_index("c")
    sid = lax.axis_index("s")
    wid = cid * NS + sid
    SEG = D // H
    QH = SEG // L
    zvec = jnp.zeros((L,), jnp.float32)

    idxbuf = ((srcS0, dstS0), (srcS1, dstS1))

    def _zrow(i, c):
        for q in range(D // L):
            zrow_v[i, pl.ds(q * L, L)] = zvec
        zs_v[i, :] = zvec
        return c
    lax.fori_loop(0, ZR, _zrow, 0)

    def _zcp(k, c):
        pltpu.sync_copy(zrow_v, acc_sh.at[pl.ds(sid * RPT + k * ZR, ZR)])
        pltpu.sync_copy(zs_v, s_sh.at[pl.ds(sid * RPT + k * ZR, ZR)])
        return c
    lax.fori_loop(0, RPT // ZR, _zcp, 0)
    plsc.subcore_barrier()

    def idx_issue(s, srcS, dstS):
        base = wid * EPT + s * SUP
        pltpu.async_copy(src_hbm.at[pl.ds(base, SUP)], srcS, s_i)
        pltpu.async_copy(dst_hbm.at[pl.ds(base, SUP)], dstS, s_i)

    def idx_wait(srcS, dstS):
        pltpu.make_async_copy(src_hbm.at[pl.ds(0, SUP)], srcS, s_i).wait()
        pltpu.make_async_copy(dst_hbm.at[pl.ds(0, SUP)], dstS, s_i).wait()

    def issue(j, srcS, dstS, buf):
        rows_v, as_v, ad_v, sem_r, sem_a, sem_b = buf
        sv = srcS.at[pl.ds(j * C, C)]
        dv = dstS.at[pl.ds(j * C, C)]
        pltpu.async_copy(h_hbm.at[sv], rows_v, sem_r)
        pltpu.async_copy(ts_hbm.at[sv], as_v, sem_a)
        pltpu.async_copy(td_hbm.at[dv], ad_v, sem_b)

    def process(j, srcS, dstS, buf):
        rows_v, as_v, ad_v, sem_r, sem_a, sem_b = buf
        sv = srcS.at[pl.ds(j * C, C)]
        dv = dstS.at[pl.ds(j * C, C)]
        pltpu.make_async_copy(ts_hbm.at[sv], as_v, sem_a).wait()
        pltpu.make_async_copy(td_hbm.at[dv], ad_v, sem_b).wait()
        pltpu.make_async_copy(h_hbm.at[sv], rows_v, sem_r).wait()

        hvs = [jnp.full((L,), h, jnp.int32) for h in range(H)]

        def _es(i, cc):
            for u in range(2):
                e = 2 * i + u
                lg = as_v[e, :] + ad_v[e, :]
                lg = jnp.maximum(lg, 0.2 * lg)
                w_v[e, :] = jnp.exp(lg)
                ev = jnp.full((L,), e, jnp.int32)
                for h in range(H):
                    wv = plsc.load_gather(w_v, [ev, hvs[h]])
                    for q in range(QH):
                        col = h * SEG + q * L
                        rows_v[e, pl.ds(col, L)] = rows_v[e, pl.ds(col, L)] * wv
            return cc
        lax.fori_loop(0, C // 2, _es, 0)

        pltpu.sync_copy(rows_v, acc_sh.at[dv], add=True)
        pltpu.sync_copy(w_v, s_sh.at[dv], add=True)

    bufA = (rowsA, asA, adA, sAr, sAa, sAb)
    bufB = (rowsB, asB, adB, sBr, sBa, sBb)
    idx_issue(0, *idxbuf[0])
    idx_wait(*idxbuf[0])
    for s in range(NSUP):
        srcS, dstS = idxbuf[s % 2]
        if s + 1 < NSUP:
            idx_issue(s + 1, *idxbuf[(s + 1) % 2])
        issue(0, srcS, dstS, bufA)

        def _pair(i, c):
            issue(2 * i + 1, srcS, dstS, bufB)
            process(2 * i, srcS, dstS, bufA)
            issue(2 * i + 2, srcS, dstS, bufA)
            process(2 * i + 1, srcS, dstS, bufB)
            return c
        lax.fori_loop(0, (NCHS - 1) // 2, _pair, 0)
        process(NCHS - 1, srcS, dstS, bufA)
        if s + 1 < NSUP:
            idx_wait(*idxbuf[(s + 1) % 2])

    plsc.subcore_barrier()

    pltpu.sync_copy(acc_sh.at[pl.ds(sid * RPT, RPT)],
                    acc_out.at[cid, pl.ds(sid * RPT, RPT)])
    pltpu.sync_copy(s_sh.at[pl.ds(sid * RPT, RPT)],
                    s_out.at[cid, pl.ds(sid * RPT, RPT)])


@functools.lru_cache(maxsize=None)
def _make_edge(D, H):
    mesh = plsc.VectorSubcoreMesh(core_axis_name="c", subcore_axis_name="s",
                                  num_cores=NC, num_subcores=NS)
    return pl.kernel(
        functools.partial(_edge_body, D, H),
        compiler_params=pltpu.CompilerParams(needs_layout_passes=False,
                                             use_tc_tiling_on_sc=False),
        out_type=[
            jax.ShapeDtypeStruct((NC, NPAD, D), jnp.float32),
            jax.ShapeDtypeStruct((NC, NPAD, SW), jnp.float32),
        ],
        mesh=mesh,
        scratch_types=[
            pltpu.VMEM((SUP,), jnp.int32),
            pltpu.VMEM((SUP,), jnp.int32),
            pltpu.VMEM((SUP,), jnp.int32),
            pltpu.VMEM((SUP,), jnp.int32),
            pltpu.VMEM((C, D), jnp.float32),
            pltpu.VMEM((C, AW), jnp.float32),
            pltpu.VMEM((C, AW), jnp.float32),
            pltpu.VMEM((C, D), jnp.float32),
            pltpu.VMEM((C, AW), jnp.float32),
            pltpu.VMEM((C, AW), jnp.float32),
            pltpu.VMEM((C, SW), jnp.float32),
            pltpu.VMEM((ZR, D), jnp.float32),
            pltpu.VMEM((ZR, SW), jnp.float32),
            pltpu.VMEM_SHARED((NPAD, D), jnp.float32),
            pltpu.VMEM_SHARED((NPAD, SW), jnp.float32),
            pltpu.SemaphoreType.DMA,
            pltpu.SemaphoreType.DMA,
            pltpu.SemaphoreType.DMA,
            pltpu.SemaphoreType.DMA,
            pltpu.SemaphoreType.DMA,
            pltpu.SemaphoreType.DMA,
            pltpu.SemaphoreType.DMA,
        ],
    )


def kernel(x, edge_index, W1, as1, ad1, b1, g1, be1, W2, as2, ad2, b2, g2,
           be2, W3, as3, ad3, b3):
    f32 = jnp.float32
    src = edge_index[0].astype(jnp.int32)
    dst = edge_index[1].astype(jnp.int32)

    eye = jnp.eye(HEADS, dtype=f32)
    zpad = jnp.zeros((HEADS * HID, AW - HEADS), f32)
    A1 = jnp.concatenate(
        [(eye[:, None, :] * as1[:, :, None]).reshape(HEADS * HID, HEADS), zpad,
         (eye[:, None, :] * ad1[:, :, None]).reshape(HEADS * HID, HEADS), zpad],
        axis=1)
    A2 = jnp.concatenate(
        [(eye[:, None, :] * as2[:, :, None]).reshape(HEADS * HID, HEADS), zpad,
         (eye[:, None, :] * ad2[:, :, None]).reshape(HEADS * HID, HEADS), zpad],
        axis=1)
    zpad3 = jnp.zeros((D_OUT, AW - 1), f32)
    A3 = jnp.concatenate([as3[0][:, None], zpad3, ad3[0][:, None], zpad3],
                         axis=1)
    P = jnp.repeat(eye, HID, axis=1)

    b1r = b1.reshape(1, -1)
    g1r = g1.reshape(1, -1)
    be1r = be1.reshape(1, -1)
    b2r = b2.reshape(1, -1)
    g2r = g2.reshape(1, -1)
    be2r = be2.reshape(1, -1)
    b3r = b3.reshape(1, -1)

    edge128 = _make_edge(HEADS * HID, HEADS)
    edge64 = _make_edge(D_OUT, 1)

    h1, t1 = _dense_in(x, W1, A1)
    acc, s = edge128(src, dst, h1, t1[:, :AW], t1[:, AW:])
    h2, t2 = _dense_mid(acc, s, b1r, g1r, be1r, W2, A2, P)
    acc, s = edge128(src, dst, h2, t2[:, :AW], t2[:, AW:])
    h3, t3 = _dense_mid(acc, s, b2r, g2r, be2r, W3, A3, P)
    acc, s = edge64(src, dst, h3, t3[:, :AW], t3[:, AW:])
    return _dense_out(acc, s, b3r)
